# Initial kernel scaffold; baseline (speedup 1.0000x reference)
import jax
import jax.numpy as jnp
from jax.experimental import pallas as pl


def kernel(x, edge_index, deg, T, raw_w, alpha):
    raise NotImplementedError("write your pallas kernel here")


# speedup vs baseline: 6.3290x; 6.3290x over previous
import jax
import jax.numpy as jnp
from jax import lax
from jax.experimental import pallas as pl
from jax.experimental.pallas import tpu as pltpu
from jax.experimental.pallas import tpu_sc as plsc

N = 10000
E = 160000
D = 16

NC = 2
NS = 16
NW = NC * NS
CHUNK = 1000
NCHUNK = E // CHUNK
CPW = NCHUNK // NW
ROWS_PER_TILE = N // NS

B2 = 2000
G2 = E // B2
B4 = 2000


def _gather_body(x_hbm, ei_hbm, xs_hbm, xd_hbm, idx_v, rows_v, sem):
    cid = lax.axis_index("c")
    sid = lax.axis_index("s")
    wid = sid * NC + cid

    def body(j, carry):
        k = wid * CPW + j
        base = k * CHUNK
        pltpu.sync_copy(ei_hbm.at[0, k], idx_v)
        pltpu.async_copy(x_hbm.at[idx_v], rows_v, sem).wait()
        pltpu.sync_copy(rows_v, xs_hbm.at[pl.ds(base, CHUNK)])
        pltpu.sync_copy(ei_hbm.at[1, k], idx_v)
        pltpu.async_copy(x_hbm.at[idx_v], rows_v, sem).wait()
        pltpu.sync_copy(rows_v, xd_hbm.at[pl.ds(base, CHUNK)])
        return carry

    lax.fori_loop(0, CPW, body, 0)


def _scatter_body(vf_hbm, vr_hbm, ei_hbm, z_hbm, out_hbm, idx_v, val_v, sem,
                  acc_sh):
    cid = lax.axis_index("c")
    sid = lax.axis_index("s")
    wid = sid * NC + cid

    row0 = sid * ROWS_PER_TILE
    pltpu.sync_copy(z_hbm.at[pl.ds(row0, ROWS_PER_TILE)],
                    acc_sh.at[pl.ds(row0, ROWS_PER_TILE)])
    plsc.subcore_barrier()

    def body(j, carry):
        k = wid * CPW + j
        base = k * CHUNK
        pltpu.sync_copy(ei_hbm.at[1, k], idx_v)
        pltpu.sync_copy(vf_hbm.at[pl.ds(base, CHUNK)], val_v)
        pltpu.sync_copy(val_v, acc_sh.at[idx_v], add=True)
        pltpu.sync_copy(ei_hbm.at[0, k], idx_v)
        pltpu.sync_copy(vr_hbm.at[pl.ds(base, CHUNK)], val_v)
        pltpu.sync_copy(val_v, acc_sh.at[idx_v], add=True)
        return carry

    lax.fori_loop(0, CPW, body, 0)
    plsc.subcore_barrier()
    pltpu.sync_copy(acc_sh.at[pl.ds(row0, ROWS_PER_TILE)],
                    out_hbm.at[cid, pl.ds(row0, ROWS_PER_TILE)])


def _softplus(x):
    return jnp.maximum(x, 0.0) + jnp.log1p(jnp.exp(-jnp.abs(x)))


def _edge_body(alpha_ref, tf_ref, tr_ref, wf_ref, wr_ref, xs_ref, xd_ref,
               vf_ref, vr_ref):
    alpha = alpha_ref[0, 0]
    r1 = lax.broadcasted_iota(jnp.int32, (D, D * D), 0)
    c1 = lax.broadcasted_iota(jnp.int32, (D, D * D), 1)
    gm = jnp.where(c1 % D == r1, 1.0, 0.0)
    r2 = lax.broadcasted_iota(jnp.int32, (D * D, D), 0)
    c2 = lax.broadcasted_iota(jnp.int32, (D * D, D), 1)
    sm = jnp.where(r2 // D == c2, 1.0, 0.0)

    xs = xs_ref[...]
    xd = xd_ref[...]

    def one_dir(t, xin, xt, rw):
        w = alpha * _softplus(rw)
        xe = jnp.dot(xin, gm, preferred_element_type=jnp.float32)
        m = jnp.dot(t * xe, sm, preferred_element_type=jnp.float32)
        return w * (m - xt)

    vf_ref[...] = one_dir(tf_ref[...], xs, xd, wf_ref[...])
    vr_ref[...] = one_dir(tr_ref[...], xd, xs, wr_ref[...])


def _combine_body(x_ref, deg_ref, p0_ref, p1_ref, out_ref):
    degf = deg_ref[...].astype(jnp.float32)
    r = 1.0 / jnp.maximum(degf, 1.0)
    out_ref[...] = x_ref[...] + r * (p0_ref[0] + p1_ref[0])


def kernel(x, edge_index, deg, T, raw_w, alpha):
    ei3 = edge_index.reshape(2, NCHUNK, CHUNK)
    rw2 = raw_w.reshape(2 * E, 1)
    deg2 = deg.reshape(N, 1)
    t_flat = T.reshape(2 * E, D * D)
    zeros_nd = jnp.zeros((N, D), jnp.float32)
    alpha2 = jnp.reshape(alpha, (1, 1))

    mesh = plsc.VectorSubcoreMesh(core_axis_name="c", subcore_axis_name="s")

    gather = pl.kernel(
        _gather_body,
        out_type=(jax.ShapeDtypeStruct((E, D), jnp.float32),
                  jax.ShapeDtypeStruct((E, D), jnp.float32)),
        mesh=mesh,
        compiler_params=pltpu.CompilerParams(use_tc_tiling_on_sc=False),
        scratch_types=[
            pltpu.VMEM((CHUNK,), jnp.int32),
            pltpu.VMEM((CHUNK, D), jnp.float32),
            pltpu.SemaphoreType.DMA,
        ],
    )
    xs, xd = gather(x, ei3)

    vf, vr = pl.pallas_call(
        _edge_body,
        grid=(G2,),
        in_specs=[
            pl.BlockSpec(memory_space=pltpu.SMEM),
            pl.BlockSpec((B2, D * D), lambda i: (i, 0)),
            pl.BlockSpec((B2, D * D), lambda i: (i + G2, 0)),
            pl.BlockSpec((B2, 1), lambda i: (i, 0)),
            pl.BlockSpec((B2, 1), lambda i: (i + G2, 0)),
            pl.BlockSpec((B2, D), lambda i: (i, 0)),
            pl.BlockSpec((B2, D), lambda i: (i, 0)),
        ],
        out_specs=[
            pl.BlockSpec((B2, D), lambda i: (i, 0)),
            pl.BlockSpec((B2, D), lambda i: (i, 0)),
        ],
        out_shape=[
            jax.ShapeDtypeStruct((E, D), jnp.float32),
            jax.ShapeDtypeStruct((E, D), jnp.float32),
        ],
    )(alpha2, t_flat, t_flat, rw2, rw2, xs, xd)

    scatter = pl.kernel(
        _scatter_body,
        out_type=jax.ShapeDtypeStruct((NC, N, D), jnp.float32),
        mesh=mesh,
        compiler_params=pltpu.CompilerParams(use_tc_tiling_on_sc=False),
        scratch_types=[
            pltpu.VMEM((CHUNK,), jnp.int32),
            pltpu.VMEM((CHUNK, D), jnp.float32),
            pltpu.SemaphoreType.DMA,
            pltpu.VMEM_SHARED((N, D), jnp.float32),
        ],
    )
    p = scatter(vf, vr, ei3, zeros_nd)

    out = pl.pallas_call(
        _combine_body,
        grid=(N // B4,),
        in_specs=[
            pl.BlockSpec((B4, D), lambda i: (i, 0)),
            pl.BlockSpec((B4, 1), lambda i: (i, 0)),
            pl.BlockSpec((1, B4, D), lambda i: (0, i, 0)),
            pl.BlockSpec((1, B4, D), lambda i: (1, i, 0)),
        ],
        out_specs=pl.BlockSpec((B4, D), lambda i: (i, 0)),
        out_shape=jax.ShapeDtypeStruct((N, D), jnp.float32),
    )(x, deg2, p, p)
    return out


# speedup vs baseline: 11.5110x; 1.8188x over previous
import jax
import jax.numpy as jnp
from jax import lax
from jax.experimental import pallas as pl
from jax.experimental.pallas import tpu as pltpu
from jax.experimental.pallas import tpu_sc as plsc

N = 10000
E = 160000
D = 16

NC = 2
NS = 16
NW = NC * NS
CHUNK = 1000
NCHUNK = E // CHUNK
CPW = NCHUNK // NW
ROWS_PER_TILE = N // NS

B2 = 3200
G2 = E // B2
B4 = 2000


def _gather_body(x_hbm, ei_hbm, xs_hbm, xd_hbm, idx_v, rows_v, sem):
    cid = lax.axis_index("c")
    sid = lax.axis_index("s")
    wid = sid * NC + cid

    def body(j, carry):
        k = wid * CPW + j
        base = k * CHUNK
        pltpu.sync_copy(ei_hbm.at[0, k], idx_v)
        pltpu.async_copy(x_hbm.at[idx_v], rows_v, sem).wait()
        pltpu.sync_copy(rows_v, xs_hbm.at[pl.ds(base, CHUNK)])
        pltpu.sync_copy(ei_hbm.at[1, k], idx_v)
        pltpu.async_copy(x_hbm.at[idx_v], rows_v, sem).wait()
        pltpu.sync_copy(rows_v, xd_hbm.at[pl.ds(base, CHUNK)])
        return carry

    lax.fori_loop(0, CPW, body, 0)


def _scatter_body(vf_hbm, vr_hbm, ei_hbm, z_hbm, out_hbm, idx_v, val_v, sem,
                  acc_sh):
    cid = lax.axis_index("c")
    sid = lax.axis_index("s")
    wid = sid * NC + cid

    row0 = sid * ROWS_PER_TILE
    pltpu.sync_copy(z_hbm.at[pl.ds(row0, ROWS_PER_TILE)],
                    acc_sh.at[pl.ds(row0, ROWS_PER_TILE)])
    plsc.subcore_barrier()

    def body(j, carry):
        k = wid * CPW + j
        base = k * CHUNK
        pltpu.sync_copy(ei_hbm.at[1, k], idx_v)
        pltpu.sync_copy(vf_hbm.at[pl.ds(base, CHUNK)], val_v)
        pltpu.sync_copy(val_v, acc_sh.at[idx_v], add=True)
        pltpu.sync_copy(ei_hbm.at[0, k], idx_v)
        pltpu.sync_copy(vr_hbm.at[pl.ds(base, CHUNK)], val_v)
        pltpu.sync_copy(val_v, acc_sh.at[idx_v], add=True)
        return carry

    lax.fori_loop(0, CPW, body, 0)
    plsc.subcore_barrier()
    pltpu.sync_copy(acc_sh.at[pl.ds(row0, ROWS_PER_TILE)],
                    out_hbm.at[cid, pl.ds(row0, ROWS_PER_TILE)])


def _softplus(x):
    return jnp.maximum(x, 0.0) + jnp.log1p(jnp.exp(-jnp.abs(x)))


def _edge_body(alpha_ref, tf_ref, tr_ref, wf_ref, wr_ref, xs_ref, xd_ref,
               vf_ref, vr_ref):
    alpha = alpha_ref[0, 0]
    r1 = lax.broadcasted_iota(jnp.int32, (D * D, D), 0)
    c1 = lax.broadcasted_iota(jnp.int32, (D * D, D), 1)
    gm2 = jnp.where(r1 % D == c1, 1.0, 0.0)
    r2 = lax.broadcasted_iota(jnp.int32, (D, D * D), 0)
    c2 = lax.broadcasted_iota(jnp.int32, (D, D * D), 1)
    sm2 = jnp.where(c2 // D == r2, 1.0, 0.0)
    r3 = lax.broadcasted_iota(jnp.int32, (D, D), 0)
    c3 = lax.broadcasted_iota(jnp.int32, (D, D), 1)
    eye = jnp.where(r3 == c3, 1.0, 0.0)

    nt = (((1,), (1,)), ((), ()))
    tn = (((0,), (0,)), ((), ()))

    def one_dir(tt, xin, xt, rw):
        w = alpha * _softplus(rw)
        xe = lax.dot_general(gm2, xin, nt, preferred_element_type=jnp.float32)
        m = jnp.dot(sm2, tt * xe, preferred_element_type=jnp.float32)
        xt_t = lax.dot_general(eye, xt, nt, preferred_element_type=jnp.float32)
        val_t = w * (m - xt_t)
        return lax.dot_general(val_t, eye, tn,
                               preferred_element_type=jnp.float32)

    vf_ref[...] = one_dir(tf_ref[...], xs_ref[...], xd_ref[...], wf_ref[0])
    vr_ref[...] = one_dir(tr_ref[...], xd_ref[...], xs_ref[...], wr_ref[0])


def _combine_body(x_ref, deg_ref, p0_ref, p1_ref, out_ref):
    degf = deg_ref[...].astype(jnp.float32)
    r = 1.0 / jnp.maximum(degf, 1.0)
    out_ref[...] = x_ref[...] + r * (p0_ref[0] + p1_ref[0])


def kernel(x, edge_index, deg, T, raw_w, alpha):
    ei3 = edge_index.reshape(2, NCHUNK, CHUNK)
    rw2 = raw_w.reshape(2 * E, 1)
    deg2 = deg.reshape(N, 1)
    t_cols = T.transpose(1, 2, 0).reshape(D * D, 2 * E)
    rw3 = raw_w.reshape(2 * G2, 1, B2)
    zeros_nd = jnp.zeros((N, D), jnp.float32)
    alpha2 = jnp.reshape(alpha, (1, 1))

    mesh = plsc.VectorSubcoreMesh(core_axis_name="c", subcore_axis_name="s")

    gather = pl.kernel(
        _gather_body,
        out_type=(jax.ShapeDtypeStruct((E, D), jnp.float32),
                  jax.ShapeDtypeStruct((E, D), jnp.float32)),
        mesh=mesh,
        compiler_params=pltpu.CompilerParams(use_tc_tiling_on_sc=False),
        scratch_types=[
            pltpu.VMEM((CHUNK,), jnp.int32),
            pltpu.VMEM((CHUNK, D), jnp.float32),
            pltpu.SemaphoreType.DMA,
        ],
    )
    xs, xd = gather(x, ei3)

    vf, vr = pl.pallas_call(
        _edge_body,
        grid=(G2,),
        in_specs=[
            pl.BlockSpec(memory_space=pltpu.SMEM),
            pl.BlockSpec((D * D, B2), lambda i: (0, i)),
            pl.BlockSpec((D * D, B2), lambda i: (0, i + G2)),
            pl.BlockSpec((1, 1, B2), lambda i: (i, 0, 0)),
            pl.BlockSpec((1, 1, B2), lambda i: (i + G2, 0, 0)),
            pl.BlockSpec((B2, D), lambda i: (i, 0)),
            pl.BlockSpec((B2, D), lambda i: (i, 0)),
        ],
        out_specs=[
            pl.BlockSpec((B2, D), lambda i: (i, 0)),
            pl.BlockSpec((B2, D), lambda i: (i, 0)),
        ],
        out_shape=[
            jax.ShapeDtypeStruct((E, D), jnp.float32),
            jax.ShapeDtypeStruct((E, D), jnp.float32),
        ],
    )(alpha2, t_cols, t_cols, rw3, rw3, xs, xd)

    scatter = pl.kernel(
        _scatter_body,
        out_type=jax.ShapeDtypeStruct((NC, N, D), jnp.float32),
        mesh=mesh,
        compiler_params=pltpu.CompilerParams(use_tc_tiling_on_sc=False),
        scratch_types=[
            pltpu.VMEM((CHUNK,), jnp.int32),
            pltpu.VMEM((CHUNK, D), jnp.float32),
            pltpu.SemaphoreType.DMA,
            pltpu.VMEM_SHARED((N, D), jnp.float32),
        ],
    )
    p = scatter(vf, vr, ei3, zeros_nd)

    out = pl.pallas_call(
        _combine_body,
        grid=(N // B4,),
        in_specs=[
            pl.BlockSpec((B4, D), lambda i: (i, 0)),
            pl.BlockSpec((B4, 1), lambda i: (i, 0)),
            pl.BlockSpec((1, B4, D), lambda i: (0, i, 0)),
            pl.BlockSpec((1, B4, D), lambda i: (1, i, 0)),
        ],
        out_specs=pl.BlockSpec((B4, D), lambda i: (i, 0)),
        out_shape=jax.ShapeDtypeStruct((N, D), jnp.float32),
    )(x, deg2, p, p)
    return out


# speedup vs baseline: 11.6097x; 1.0086x over previous
import jax
import jax.numpy as jnp
from jax import lax
from jax.experimental import pallas as pl
from jax.experimental.pallas import tpu as pltpu
from jax.experimental.pallas import tpu_sc as plsc

N = 10000
E = 160000
D = 16

NC = 2
NS = 16
NW = NC * NS
CHUNK = 1000
NCHUNK = E // CHUNK
CPW = NCHUNK // NW
ROWS_PER_TILE = N // NS

B2 = 3200
G2 = E // B2
B4 = 2000


def _gather_body(x_hbm, ei_hbm, xs_hbm, xd_hbm, idx_v, rows_v, sem):
    cid = lax.axis_index("c")
    sid = lax.axis_index("s")
    wid = sid * NC + cid

    def body(j, carry):
        k = wid * CPW + j
        base = k * CHUNK
        pltpu.sync_copy(ei_hbm.at[0, k], idx_v)
        pltpu.async_copy(x_hbm.at[idx_v], rows_v, sem).wait()
        pltpu.sync_copy(rows_v, xs_hbm.at[pl.ds(base, CHUNK)])
        pltpu.sync_copy(ei_hbm.at[1, k], idx_v)
        pltpu.async_copy(x_hbm.at[idx_v], rows_v, sem).wait()
        pltpu.sync_copy(rows_v, xd_hbm.at[pl.ds(base, CHUNK)])
        return carry

    lax.fori_loop(0, CPW, body, 0)


def _scatter_body(vf_hbm, vr_hbm, ei_hbm, z_hbm, out_hbm, idx_v, val_v, sem,
                  acc_sh):
    cid = lax.axis_index("c")
    sid = lax.axis_index("s")
    wid = sid * NC + cid

    row0 = sid * ROWS_PER_TILE
    pltpu.sync_copy(z_hbm.at[pl.ds(row0, ROWS_PER_TILE)],
                    acc_sh.at[pl.ds(row0, ROWS_PER_TILE)])
    plsc.subcore_barrier()

    def body(j, carry):
        k = wid * CPW + j
        base = k * CHUNK
        pltpu.sync_copy(ei_hbm.at[1, k], idx_v)
        pltpu.sync_copy(vf_hbm.at[pl.ds(base, CHUNK)], val_v)
        pltpu.sync_copy(val_v, acc_sh.at[idx_v], add=True)
        pltpu.sync_copy(ei_hbm.at[0, k], idx_v)
        pltpu.sync_copy(vr_hbm.at[pl.ds(base, CHUNK)], val_v)
        pltpu.sync_copy(val_v, acc_sh.at[idx_v], add=True)
        return carry

    lax.fori_loop(0, CPW, body, 0)
    plsc.subcore_barrier()
    pltpu.sync_copy(acc_sh.at[pl.ds(row0, ROWS_PER_TILE)],
                    out_hbm.at[cid, pl.ds(row0, ROWS_PER_TILE)])


def _softplus(x):
    return jnp.maximum(x, 0.0) + jnp.log1p(jnp.exp(-jnp.abs(x)))


def _edge_body(alpha_ref, tf_ref, tr_ref, wf_ref, wr_ref, xs_ref, xd_ref,
               vf_ref, vr_ref):
    alpha = alpha_ref[0, 0]
    r1 = lax.broadcasted_iota(jnp.int32, (D * D, D), 0)
    c1 = lax.broadcasted_iota(jnp.int32, (D * D, D), 1)
    gm2 = jnp.where(r1 % D == c1, 1.0, 0.0)
    r2 = lax.broadcasted_iota(jnp.int32, (D, D * D), 0)
    c2 = lax.broadcasted_iota(jnp.int32, (D, D * D), 1)
    sm2 = jnp.where(c2 // D == r2, 1.0, 0.0)

    nt = (((1,), (1,)), ((), ()))

    xs = xs_ref[...]
    xd = xd_ref[...]

    def one_dir(tt, xin, xt, rw):
        w = alpha * _softplus(rw)
        xe = lax.dot_general(gm2, xin, nt, preferred_element_type=jnp.float32)
        m = jnp.dot(sm2, tt * xe, preferred_element_type=jnp.float32)
        val_t = w * (m - jnp.transpose(xt))
        return jnp.transpose(val_t)

    vf_ref[...] = one_dir(tf_ref[...], xs, xd, wf_ref[0])
    vr_ref[...] = one_dir(tr_ref[...], xd, xs, wr_ref[0])


def _combine_body(x_ref, deg_ref, p0_ref, p1_ref, out_ref):
    degf = deg_ref[...].astype(jnp.float32)
    r = 1.0 / jnp.maximum(degf, 1.0)
    out_ref[...] = x_ref[...] + r * (p0_ref[0] + p1_ref[0])


def kernel(x, edge_index, deg, T, raw_w, alpha):
    ei3 = edge_index.reshape(2, NCHUNK, CHUNK)
    t_cols = T.transpose(1, 2, 0).reshape(D * D, 2 * E)
    rw3 = raw_w.reshape(2 * G2, 1, B2)
    deg2 = deg.reshape(N, 1)
    zeros_nd = jnp.zeros((N, D), jnp.float32)
    alpha2 = jnp.reshape(alpha, (1, 1))

    mesh = plsc.VectorSubcoreMesh(core_axis_name="c", subcore_axis_name="s")

    gather = pl.kernel(
        _gather_body,
        out_type=(jax.ShapeDtypeStruct((E, D), jnp.float32),
                  jax.ShapeDtypeStruct((E, D), jnp.float32)),
        mesh=mesh,
        compiler_params=pltpu.CompilerParams(use_tc_tiling_on_sc=False),
        scratch_types=[
            pltpu.VMEM((CHUNK,), jnp.int32),
            pltpu.VMEM((CHUNK, D), jnp.float32),
            pltpu.SemaphoreType.DMA,
        ],
    )
    xs, xd = gather(x, ei3)

    vf, vr = pl.pallas_call(
        _edge_body,
        grid=(G2,),
        in_specs=[
            pl.BlockSpec(memory_space=pltpu.SMEM),
            pl.BlockSpec((D * D, B2), lambda i: (0, i)),
            pl.BlockSpec((D * D, B2), lambda i: (0, i + G2)),
            pl.BlockSpec((1, 1, B2), lambda i: (i, 0, 0)),
            pl.BlockSpec((1, 1, B2), lambda i: (i + G2, 0, 0)),
            pl.BlockSpec((B2, D), lambda i: (i, 0)),
            pl.BlockSpec((B2, D), lambda i: (i, 0)),
        ],
        out_specs=[
            pl.BlockSpec((B2, D), lambda i: (i, 0)),
            pl.BlockSpec((B2, D), lambda i: (i, 0)),
        ],
        out_shape=[
            jax.ShapeDtypeStruct((E, D), jnp.float32),
            jax.ShapeDtypeStruct((E, D), jnp.float32),
        ],
    )(alpha2, t_cols, t_cols, rw3, rw3, xs, xd)

    scatter = pl.kernel(
        _scatter_body,
        out_type=jax.ShapeDtypeStruct((NC, N, D), jnp.float32),
        mesh=mesh,
        compiler_params=pltpu.CompilerParams(use_tc_tiling_on_sc=False),
        scratch_types=[
            pltpu.VMEM((CHUNK,), jnp.int32),
            pltpu.VMEM((CHUNK, D), jnp.float32),
            pltpu.SemaphoreType.DMA,
            pltpu.VMEM_SHARED((N, D), jnp.float32),
        ],
    )
    p = scatter(vf, vr, ei3, zeros_nd)

    out = pl.pallas_call(
        _combine_body,
        grid=(N // B4,),
        in_specs=[
            pl.BlockSpec((B4, D), lambda i: (i, 0)),
            pl.BlockSpec((B4, 1), lambda i: (i, 0)),
            pl.BlockSpec((1, B4, D), lambda i: (0, i, 0)),
            pl.BlockSpec((1, B4, D), lambda i: (1, i, 0)),
        ],
        out_specs=pl.BlockSpec((B4, D), lambda i: (i, 0)),
        out_shape=jax.ShapeDtypeStruct((N, D), jnp.float32),
    )(x, deg2, p, p)
    return out


# speedup vs baseline: 12.0807x; 1.0406x over previous
import jax
import jax.numpy as jnp
from jax import lax
from jax.experimental import pallas as pl
from jax.experimental.pallas import tpu as pltpu
from jax.experimental.pallas import tpu_sc as plsc

N = 10000
E = 160000
D = 16

NC = 2
NS = 16
NW = NC * NS
CHUNK = 5000
NCHUNK = E // CHUNK
CPW = NCHUNK // NW
ROWS_PER_TILE = N // NS

B2 = 6400
G2 = E // B2
B4 = 2000


def _gather_body(x_hbm, ei_hbm, xs_hbm, xd_hbm, idx_v, rows_v, sem):
    cid = lax.axis_index("c")
    sid = lax.axis_index("s")
    wid = sid * NC + cid

    def body(j, carry):
        k = wid * CPW + j
        base = k * CHUNK
        pltpu.sync_copy(ei_hbm.at[0, k], idx_v)
        pltpu.async_copy(x_hbm.at[idx_v], rows_v, sem).wait()
        pltpu.sync_copy(rows_v, xs_hbm.at[pl.ds(base, CHUNK)])
        pltpu.sync_copy(ei_hbm.at[1, k], idx_v)
        pltpu.async_copy(x_hbm.at[idx_v], rows_v, sem).wait()
        pltpu.sync_copy(rows_v, xd_hbm.at[pl.ds(base, CHUNK)])
        return carry

    lax.fori_loop(0, CPW, body, 0)


def _scatter_body(vf_hbm, vr_hbm, ei_hbm, z_hbm, out_hbm, idx_v, val_v, sem,
                  acc_sh):
    cid = lax.axis_index("c")
    sid = lax.axis_index("s")
    wid = sid * NC + cid

    row0 = sid * ROWS_PER_TILE
    pltpu.sync_copy(z_hbm.at[pl.ds(row0, ROWS_PER_TILE)],
                    acc_sh.at[pl.ds(row0, ROWS_PER_TILE)])
    plsc.subcore_barrier()

    def body(j, carry):
        k = wid * CPW + j
        base = k * CHUNK
        pltpu.sync_copy(ei_hbm.at[1, k], idx_v)
        pltpu.sync_copy(vf_hbm.at[pl.ds(base, CHUNK)], val_v)
        pltpu.sync_copy(val_v, acc_sh.at[idx_v], add=True)
        pltpu.sync_copy(ei_hbm.at[0, k], idx_v)
        pltpu.sync_copy(vr_hbm.at[pl.ds(base, CHUNK)], val_v)
        pltpu.sync_copy(val_v, acc_sh.at[idx_v], add=True)
        return carry

    lax.fori_loop(0, CPW, body, 0)
    plsc.subcore_barrier()
    pltpu.sync_copy(acc_sh.at[pl.ds(row0, ROWS_PER_TILE)],
                    out_hbm.at[cid, pl.ds(row0, ROWS_PER_TILE)])


def _softplus(x):
    return jnp.maximum(x, 0.0) + jnp.log1p(jnp.exp(-jnp.abs(x)))


def _edge_body(alpha_ref, tf_ref, tr_ref, wf_ref, wr_ref, xs_ref, xd_ref,
               vf_ref, vr_ref):
    alpha = alpha_ref[0, 0]
    r1 = lax.broadcasted_iota(jnp.int32, (D * D, D), 0)
    c1 = lax.broadcasted_iota(jnp.int32, (D * D, D), 1)
    gm2 = jnp.where(r1 % D == c1, 1.0, 0.0)
    r2 = lax.broadcasted_iota(jnp.int32, (D, D * D), 0)
    c2 = lax.broadcasted_iota(jnp.int32, (D, D * D), 1)
    sm2 = jnp.where(c2 // D == r2, 1.0, 0.0)

    nt = (((1,), (1,)), ((), ()))

    xs = xs_ref[...]
    xd = xd_ref[...]

    def one_dir(tt, xin, xt, rw):
        w = alpha * _softplus(rw)
        xe = lax.dot_general(gm2, xin, nt, preferred_element_type=jnp.float32)
        m = jnp.dot(sm2, tt * xe, preferred_element_type=jnp.float32)
        val_t = w * (m - jnp.transpose(xt))
        return jnp.transpose(val_t)

    vf_ref[...] = one_dir(tf_ref[...], xs, xd, wf_ref[0])
    vr_ref[...] = one_dir(tr_ref[...], xd, xs, wr_ref[0])


def _combine_body(x_ref, deg_ref, p0_ref, p1_ref, out_ref):
    degf = deg_ref[0].astype(jnp.float32)
    r = jnp.transpose(1.0 / jnp.maximum(degf, 1.0))
    out_ref[...] = x_ref[...] + r * (p0_ref[0] + p1_ref[0])


def kernel(x, edge_index, deg, T, raw_w, alpha):
    ei3 = edge_index.reshape(2, NCHUNK, CHUNK)
    t_cols = T.transpose(1, 2, 0).reshape(D * D, 2 * E)
    rw3 = raw_w.reshape(2 * G2, 1, B2)
    deg3 = deg.reshape(N // B4, 1, B4)
    zeros_nd = jnp.zeros((N, D), jnp.float32)
    alpha2 = jnp.reshape(alpha, (1, 1))

    mesh = plsc.VectorSubcoreMesh(core_axis_name="c", subcore_axis_name="s")

    gather = pl.kernel(
        _gather_body,
        out_type=(jax.ShapeDtypeStruct((E, D), jnp.float32),
                  jax.ShapeDtypeStruct((E, D), jnp.float32)),
        mesh=mesh,
        compiler_params=pltpu.CompilerParams(use_tc_tiling_on_sc=False),
        scratch_types=[
            pltpu.VMEM((CHUNK,), jnp.int32),
            pltpu.VMEM((CHUNK, D), jnp.float32),
            pltpu.SemaphoreType.DMA,
        ],
    )
    xs, xd = gather(x, ei3)

    vf, vr = pl.pallas_call(
        _edge_body,
        grid=(G2,),
        in_specs=[
            pl.BlockSpec(memory_space=pltpu.SMEM),
            pl.BlockSpec((D * D, B2), lambda i: (0, i)),
            pl.BlockSpec((D * D, B2), lambda i: (0, i + G2)),
            pl.BlockSpec((1, 1, B2), lambda i: (i, 0, 0)),
            pl.BlockSpec((1, 1, B2), lambda i: (i + G2, 0, 0)),
            pl.BlockSpec((B2, D), lambda i: (i, 0)),
            pl.BlockSpec((B2, D), lambda i: (i, 0)),
        ],
        out_specs=[
            pl.BlockSpec((B2, D), lambda i: (i, 0)),
            pl.BlockSpec((B2, D), lambda i: (i, 0)),
        ],
        out_shape=[
            jax.ShapeDtypeStruct((E, D), jnp.float32),
            jax.ShapeDtypeStruct((E, D), jnp.float32),
        ],
    )(alpha2, t_cols, t_cols, rw3, rw3, xs, xd)

    scatter = pl.kernel(
        _scatter_body,
        out_type=jax.ShapeDtypeStruct((NC, N, D), jnp.float32),
        mesh=mesh,
        compiler_params=pltpu.CompilerParams(use_tc_tiling_on_sc=False),
        scratch_types=[
            pltpu.VMEM((CHUNK,), jnp.int32),
            pltpu.VMEM((CHUNK, D), jnp.float32),
            pltpu.SemaphoreType.DMA,
            pltpu.VMEM_SHARED((N, D), jnp.float32),
        ],
    )
    p = scatter(vf, vr, ei3, zeros_nd)

    out = pl.pallas_call(
        _combine_body,
        grid=(N // B4,),
        in_specs=[
            pl.BlockSpec((B4, D), lambda i: (i, 0)),
            pl.BlockSpec((1, 1, B4), lambda i: (i, 0, 0)),
            pl.BlockSpec((1, B4, D), lambda i: (0, i, 0)),
            pl.BlockSpec((1, B4, D), lambda i: (1, i, 0)),
        ],
        out_specs=pl.BlockSpec((B4, D), lambda i: (i, 0)),
        out_shape=jax.ShapeDtypeStruct((N, D), jnp.float32),
    )(x, deg3, p, p)
    return out
